# baseline (speedup 1.0000x reference)
import functools

import jax
import jax.numpy as jnp
from jax import lax
from jax.experimental import pallas as pl
from jax.experimental.pallas import tpu as pltpu
from jax.experimental.pallas import tpu_sc as plsc

_L = 16


def _entropy_pairs(x):
    lane = lax.broadcasted_iota(jnp.int32, x.shape, 1)
    even = (lane % 2) == 0
    partner = jnp.where(even, jnp.roll(x, -1, axis=1), jnp.roll(x, 1, axis=1))
    m = jnp.maximum(x, partner)
    e0 = jnp.exp(x - m)
    e1 = jnp.exp(partner - m)
    tot = e0 + e1
    po = e0 / tot
    pp = e1 / tot
    eps = 1e-10
    factor = 1.0 + 0.01 / (1.0 + 1 * 0)
    h = ((po + eps) * jnp.log(1.0 / (po + eps) + eps)
         + (pp + eps) * jnp.log(1.0 / (pp + eps) + eps))
    return h * factor


def _entropy_tc_body(xn_ref, xc_ref, hn_ref, hc_ref):
    hn_ref[...] = _entropy_pairs(xn_ref[...])
    hc_ref[...] = _entropy_pairs(xc_ref[...])


def _floor16(x):
    t = x.astype(jnp.int32).astype(jnp.float32)
    return t - jnp.where(x < t, 1.0, 0.0)


def _rmw_max(ref, idx, s):
    ks, vi = plsc.sort_key_val(s, idx)
    _, last = plsc.scan_count(vi)
    cur = plsc.load_gather(ref, [vi])
    plsc.store_scatter(ref, [vi], jnp.maximum(cur, ks), mask=last)


def _make_sc_kernels(n_nodes, n_edges):
    try:
        info = plsc.get_sparse_core_info()
        nc, ns = info.num_cores, info.num_subcores
    except ValueError:
        nc, ns = 2, 16
    nw = nc * ns
    slc = ((n_nodes + ns * _L - 1) // (ns * _L)) * _L
    n_pad = ns * slc
    chunk = ((n_edges + nw * _L - 1) // (nw * _L)) * _L
    tail = n_edges - (nw - 1) * chunk
    assert tail > 0 and tail % _L == 0 and chunk % 32 == 0
    assert n_pad % 128 == 0 and slc % _L == 0
    mesh = plsc.VectorSubcoreMesh(core_axis_name="c", subcore_axis_name="s")

    def stage_edge_chunks(ei, wid, base, src_v, dst_v):
        pltpu.sync_copy(ei.at[pl.ds(base, chunk)], src_v)

        @pl.when(wid < nw - 1)
        def _():
            pltpu.sync_copy(ei.at[pl.ds(n_edges + base, chunk)], dst_v)

        @pl.when(wid == nw - 1)
        def _():
            pltpu.sync_copy(ei.at[pl.ds(n_edges + base, tail)],
                            dst_v.at[pl.ds(0, tail)])
            izeros = jnp.zeros((_L,), jnp.int32)
            for u in range((chunk - tail) // _L):
                dst_v[pl.ds(tail + u * _L, _L)] = izeros

    @functools.partial(
        pl.kernel,
        out_type=(
            jax.ShapeDtypeStruct((nw * chunk,), jnp.float32),
            jax.ShapeDtypeStruct((nc * n_pad,), jnp.float32),
        ),
        mesh=mesh,
        compiler_params=pltpu.CompilerParams(needs_layout_passes=False),
        scratch_types=[
            pltpu.VMEM((2 * n_nodes,), jnp.float32),
            pltpu.VMEM((n_pad,), jnp.float32),
            pltpu.VMEM((chunk,), jnp.int32),
            pltpu.VMEM((chunk,), jnp.int32),
            pltpu.VMEM((2 * chunk,), jnp.float32),
            pltpu.VMEM((chunk,), jnp.float32),
            pltpu.VMEM_SHARED((ns * n_pad,), jnp.float32),
        ],
    )
    def sc1(hn, hc, ei, scores_out, nm_out, h_v, nm_v, src_v, dst_v, hc_v,
            sc_v, partials):
        cid = lax.axis_index("c")
        sid = lax.axis_index("s")
        wid = sid * nc + cid
        base = wid * chunk

        pltpu.sync_copy(hn, h_v)
        stage_edge_chunks(ei, wid, base, src_v, dst_v)

        @pl.when(wid < nw - 1)
        def _():
            pltpu.sync_copy(hc.at[pl.ds(2 * base, 2 * chunk)], hc_v)

        @pl.when(wid == nw - 1)
        def _():
            pltpu.sync_copy(hc.at[pl.ds(2 * base, 2 * tail)],
                            hc_v.at[pl.ds(0, 2 * tail)])
            fzeros = jnp.zeros((_L,), jnp.float32)
            for u in range(2 * (chunk - tail) // _L):
                hc_v[pl.ds(2 * tail + u * _L, _L)] = fzeros

        zeros = jnp.zeros((_L,), jnp.float32)

        def zero_body(j, _):
            for u in range(8):
                nm_v[pl.ds(j * 8 * _L + u * _L, _L)] = zeros
            return 0

        lax.fori_loop(0, n_pad // (8 * _L), zero_body, 0)

        iota = lax.iota(jnp.int32, _L)
        iota2 = iota + iota

        def edge_body(j, _):
            for u in range(2):
                off = (j * 2 + u) * _L
                sl = pl.ds(off, _L)
                si = src_v[sl]
                di = dst_v[sl]
                hcv = plsc.load_gather(hc_v, [2 * off + iota2])
                hs = plsc.load_gather(h_v, [si + si])
                hd = plsc.load_gather(h_v, [di + di])
                a = hs - hcv
                b = hd - hcv
                fa = _floor16(a)
                fb = _floor16(b)
                s = (2.0 + a) * (2.0 + b) * ((1.0 + fa) * (1.0 + fb))
                lane = base + off + iota
                s = jnp.where(lane < n_edges, s, 0.0)
                sc_v[sl] = s
                _rmw_max(nm_v, si, s)
                _rmw_max(nm_v, di, s)
            return 0

        lax.fori_loop(0, chunk // (2 * _L), edge_body, 0)

        pltpu.sync_copy(sc_v, scores_out.at[pl.ds(base, chunk)])

        pltpu.sync_copy(nm_v, partials.at[pl.ds(sid * n_pad, n_pad)])
        plsc.subcore_barrier()
        for t in range(ns):
            pltpu.sync_copy(partials.at[pl.ds(t * n_pad + sid * slc, slc)],
                            h_v.at[pl.ds(t * slc, slc)])

        def red_body(j, _):
            off = j * _L
            acc = h_v[pl.ds(off, _L)]
            for t in range(1, ns):
                acc = jnp.maximum(acc, h_v[pl.ds(t * slc + off, _L)])
            nm_v[pl.ds(off, _L)] = acc
            return 0

        lax.fori_loop(0, slc // _L, red_body, 0)
        pltpu.sync_copy(nm_v.at[pl.ds(0, slc)],
                        nm_out.at[pl.ds(cid * n_pad + sid * slc, slc)])

    @functools.partial(
        pl.kernel,
        out_type=jax.ShapeDtypeStruct((n_edges,), jnp.float32),
        mesh=mesh,
        compiler_params=pltpu.CompilerParams(needs_layout_passes=False),
        scratch_types=[
            pltpu.VMEM((n_pad,), jnp.float32),
            pltpu.VMEM((n_pad,), jnp.float32),
            pltpu.VMEM((chunk,), jnp.int32),
            pltpu.VMEM((chunk,), jnp.int32),
            pltpu.VMEM((chunk,), jnp.float32),
        ],
    )
    def sc2(nm_parts, ei, scores, out, nm_v, nm2_v, src_v, dst_v, sc_v):
        cid = lax.axis_index("c")
        sid = lax.axis_index("s")
        wid = sid * nc + cid
        base = wid * chunk

        pltpu.sync_copy(nm_parts.at[pl.ds(0, n_pad)], nm_v)
        pltpu.sync_copy(nm_parts.at[pl.ds(n_pad, n_pad)], nm2_v)
        stage_edge_chunks(ei, wid, base, src_v, dst_v)
        pltpu.sync_copy(scores.at[pl.ds(base, chunk)], sc_v)

        def merge_body(j, _):
            for u in range(8):
                sl = pl.ds(j * 8 * _L + u * _L, _L)
                nm_v[sl] = jnp.maximum(nm_v[sl], nm2_v[sl])
            return 0

        lax.fori_loop(0, n_pad // (8 * _L), merge_body, 0)

        def sel_body(j, _):
            for u in range(2):
                sl = pl.ds((j * 2 + u) * _L, _L)
                s = sc_v[sl]
                ms = plsc.load_gather(nm_v, [src_v[sl]])
                md = plsc.load_gather(nm_v, [dst_v[sl]])
                keep = (s > 0.0) & (s >= ms) & (s >= md)
                sc_v[sl] = jnp.where(keep, s, 0.0)
            return 0

        lax.fori_loop(0, chunk // (2 * _L), sel_body, 0)

        @pl.when(wid < nw - 1)
        def _():
            pltpu.sync_copy(sc_v, out.at[pl.ds(base, chunk)])

        @pl.when(wid == nw - 1)
        def _():
            pltpu.sync_copy(sc_v.at[pl.ds(0, tail)], out.at[pl.ds(base, tail)])

    return sc1, sc2


@jax.jit
def kernel(node_logits, comb_logits, edge_index):
    n_nodes = node_logits.shape[0]
    n_edges = comb_logits.shape[0]
    sc1, sc2 = _make_sc_kernels(n_nodes, n_edges)

    xn = node_logits.reshape(-1, 160)
    xc = comb_logits.reshape(-1, 160)
    hn2d, hc2d = pl.pallas_call(
        _entropy_tc_body,
        out_shape=(
            jax.ShapeDtypeStruct(xn.shape, jnp.float32),
            jax.ShapeDtypeStruct(xc.shape, jnp.float32),
        ),
    )(xn, xc)
    hn = hn2d.reshape(2 * n_nodes)
    hc = hc2d.reshape(2 * n_edges)
    ei = edge_index.reshape(2 * n_edges)

    scores, nm_parts = sc1(hn, hc, ei)
    return sc2(nm_parts, ei, scores)


# speedup vs baseline: 1.9465x; 1.9465x over previous
import functools

import jax
import jax.numpy as jnp
from jax import lax
from jax.experimental import pallas as pl
from jax.experimental.pallas import tpu as pltpu
from jax.experimental.pallas import tpu_sc as plsc

_L = 16


def _entropy_cols(l0, l1):
    m = jnp.maximum(l0, l1)
    e0 = jnp.exp(l0 - m)
    e1 = jnp.exp(l1 - m)
    tot = e0 + e1
    p0 = e0 / tot
    p1 = e1 / tot
    eps = 1e-10
    factor = 1.0 + 0.01 / (1.0 + 1 * 0)
    h = ((p0 + eps) * jnp.log(1.0 / (p0 + eps) + eps)
         + (p1 + eps) * jnp.log(1.0 / (p1 + eps) + eps))
    return h * factor


def _entropy_tc_body(xn_ref, xc_ref, hn_ref, hc_ref):
    hn_ref[...] = _entropy_cols(xn_ref[0, :], xn_ref[1, :])
    hc_ref[...] = _entropy_cols(xc_ref[0, :], xc_ref[1, :])


def _floor16(x):
    t = x.astype(jnp.int32).astype(jnp.float32)
    return t - jnp.where(x < t, 1.0, 0.0)


def _rmw_max(ref, idx, s):
    ks, vi = plsc.sort_key_val(s, idx)
    _, last = plsc.scan_count(vi)
    cur = plsc.load_gather(ref, [vi])
    plsc.store_scatter(ref, [vi], jnp.maximum(cur, ks), mask=last)


def _make_sc_kernels(n_nodes, n_edges):
    try:
        info = plsc.get_sparse_core_info()
        nc, ns = info.num_cores, info.num_subcores
    except ValueError:
        nc, ns = 2, 16
    nw = nc * ns
    slc = ((n_nodes + ns * _L - 1) // (ns * _L)) * _L
    n_pad = ns * slc
    chunk = ((n_edges + nw * _L - 1) // (nw * _L)) * _L
    tail = n_edges - (nw - 1) * chunk
    assert tail > 0 and tail % _L == 0 and chunk % 32 == 0
    assert n_pad % 128 == 0 and slc % _L == 0
    mesh = plsc.VectorSubcoreMesh(core_axis_name="c", subcore_axis_name="s")

    def stage_edge_chunks(ei, wid, base, src_v, dst_v):
        pltpu.sync_copy(ei.at[pl.ds(base, chunk)], src_v)

        @pl.when(wid < nw - 1)
        def _():
            pltpu.sync_copy(ei.at[pl.ds(n_edges + base, chunk)], dst_v)

        @pl.when(wid == nw - 1)
        def _():
            pltpu.sync_copy(ei.at[pl.ds(n_edges + base, tail)],
                            dst_v.at[pl.ds(0, tail)])
            izeros = jnp.zeros((_L,), jnp.int32)
            for u in range((chunk - tail) // _L):
                dst_v[pl.ds(tail + u * _L, _L)] = izeros

    @functools.partial(
        pl.kernel,
        out_type=(
            jax.ShapeDtypeStruct((nw * chunk,), jnp.float32),
            jax.ShapeDtypeStruct((nc * n_pad,), jnp.float32),
        ),
        mesh=mesh,
        compiler_params=pltpu.CompilerParams(needs_layout_passes=False),
        scratch_types=[
            pltpu.VMEM((n_pad,), jnp.float32),
            pltpu.VMEM((n_pad,), jnp.float32),
            pltpu.VMEM((chunk,), jnp.int32),
            pltpu.VMEM((chunk,), jnp.int32),
            pltpu.VMEM((chunk,), jnp.float32),
            pltpu.VMEM((chunk,), jnp.float32),
            pltpu.VMEM_SHARED((ns * n_pad,), jnp.float32),
        ],
    )
    def sc1(hn, hc, ei, scores_out, nm_out, h_v, nm_v, src_v, dst_v, hc_v,
            sc_v, partials):
        cid = lax.axis_index("c")
        sid = lax.axis_index("s")
        wid = sid * nc + cid
        base = wid * chunk

        pltpu.sync_copy(hn, h_v.at[pl.ds(0, n_nodes)])
        stage_edge_chunks(ei, wid, base, src_v, dst_v)

        @pl.when(wid < nw - 1)
        def _():
            pltpu.sync_copy(hc.at[pl.ds(base, chunk)], hc_v)

        @pl.when(wid == nw - 1)
        def _():
            pltpu.sync_copy(hc.at[pl.ds(base, tail)],
                            hc_v.at[pl.ds(0, tail)])
            fzeros = jnp.zeros((_L,), jnp.float32)
            for u in range((chunk - tail) // _L):
                hc_v[pl.ds(tail + u * _L, _L)] = fzeros

        zeros = jnp.zeros((_L,), jnp.float32)

        def zero_body(j, _):
            for u in range(8):
                nm_v[pl.ds(j * 8 * _L + u * _L, _L)] = zeros
            return 0

        lax.fori_loop(0, n_pad // (8 * _L), zero_body, 0)

        iota = lax.iota(jnp.int32, _L)

        def edge_body(j, _):
            for u in range(2):
                off = (j * 2 + u) * _L
                sl = pl.ds(off, _L)
                si = src_v[sl]
                di = dst_v[sl]
                hcv = hc_v[sl]
                hs = plsc.load_gather(h_v, [si])
                hd = plsc.load_gather(h_v, [di])
                a = hs - hcv
                b = hd - hcv
                fa = _floor16(a)
                fb = _floor16(b)
                s = (2.0 + a) * (2.0 + b) * ((1.0 + fa) * (1.0 + fb))
                lane = base + off + iota
                s = jnp.where(lane < n_edges, s, 0.0)
                sc_v[sl] = s
                _rmw_max(nm_v, si, s)
                _rmw_max(nm_v, di, s)
            return 0

        lax.fori_loop(0, chunk // (2 * _L), edge_body, 0)

        pltpu.sync_copy(sc_v, scores_out.at[pl.ds(base, chunk)])

        pltpu.sync_copy(nm_v, partials.at[pl.ds(sid * n_pad, n_pad)])
        plsc.subcore_barrier()
        for t in range(ns):
            pltpu.sync_copy(partials.at[pl.ds(t * n_pad + sid * slc, slc)],
                            h_v.at[pl.ds(t * slc, slc)])

        def red_body(j, _):
            off = j * _L
            acc = h_v[pl.ds(off, _L)]
            for t in range(1, ns):
                acc = jnp.maximum(acc, h_v[pl.ds(t * slc + off, _L)])
            nm_v[pl.ds(off, _L)] = acc
            return 0

        lax.fori_loop(0, slc // _L, red_body, 0)
        pltpu.sync_copy(nm_v.at[pl.ds(0, slc)],
                        nm_out.at[pl.ds(cid * n_pad + sid * slc, slc)])

    @functools.partial(
        pl.kernel,
        out_type=jax.ShapeDtypeStruct((n_edges,), jnp.float32),
        mesh=mesh,
        compiler_params=pltpu.CompilerParams(needs_layout_passes=False),
        scratch_types=[
            pltpu.VMEM((n_pad,), jnp.float32),
            pltpu.VMEM((n_pad,), jnp.float32),
            pltpu.VMEM((chunk,), jnp.int32),
            pltpu.VMEM((chunk,), jnp.int32),
            pltpu.VMEM((chunk,), jnp.float32),
        ],
    )
    def sc2(nm_parts, ei, scores, out, nm_v, nm2_v, src_v, dst_v, sc_v):
        cid = lax.axis_index("c")
        sid = lax.axis_index("s")
        wid = sid * nc + cid
        base = wid * chunk

        pltpu.sync_copy(nm_parts.at[pl.ds(0, n_pad)], nm_v)
        pltpu.sync_copy(nm_parts.at[pl.ds(n_pad, n_pad)], nm2_v)
        stage_edge_chunks(ei, wid, base, src_v, dst_v)
        pltpu.sync_copy(scores.at[pl.ds(base, chunk)], sc_v)

        def merge_body(j, _):
            for u in range(8):
                sl = pl.ds(j * 8 * _L + u * _L, _L)
                nm_v[sl] = jnp.maximum(nm_v[sl], nm2_v[sl])
            return 0

        lax.fori_loop(0, n_pad // (8 * _L), merge_body, 0)

        def sel_body(j, _):
            for u in range(2):
                sl = pl.ds((j * 2 + u) * _L, _L)
                s = sc_v[sl]
                ms = plsc.load_gather(nm_v, [src_v[sl]])
                md = plsc.load_gather(nm_v, [dst_v[sl]])
                keep = (s > 0.0) & (s >= ms) & (s >= md)
                sc_v[sl] = jnp.where(keep, s, 0.0)
            return 0

        lax.fori_loop(0, chunk // (2 * _L), sel_body, 0)

        @pl.when(wid < nw - 1)
        def _():
            pltpu.sync_copy(sc_v, out.at[pl.ds(base, chunk)])

        @pl.when(wid == nw - 1)
        def _():
            pltpu.sync_copy(sc_v.at[pl.ds(0, tail)], out.at[pl.ds(base, tail)])

    return sc1, sc2


@jax.jit
def kernel(node_logits, comb_logits, edge_index):
    n_nodes = node_logits.shape[0]
    n_edges = comb_logits.shape[0]
    sc1, sc2 = _make_sc_kernels(n_nodes, n_edges)

    xn = node_logits.T
    xc = comb_logits.T
    hn, hc = pl.pallas_call(
        _entropy_tc_body,
        out_shape=(
            jax.ShapeDtypeStruct((n_nodes,), jnp.float32),
            jax.ShapeDtypeStruct((n_edges,), jnp.float32),
        ),
    )(xn, xc)
    ei = edge_index.reshape(2 * n_edges)

    scores, nm_parts = sc1(hn, hc, ei)
    return sc2(nm_parts, ei, scores)


# speedup vs baseline: 2.5018x; 1.2853x over previous
import functools

import jax
import jax.numpy as jnp
from jax import lax
from jax.experimental import pallas as pl
from jax.experimental.pallas import tpu as pltpu
from jax.experimental.pallas import tpu_sc as plsc

_L = 16


def _entropy_cols(l0, l1):
    m = jnp.maximum(l0, l1)
    e0 = jnp.exp(l0 - m)
    e1 = jnp.exp(l1 - m)
    tot = e0 + e1
    p0 = e0 / tot
    p1 = e1 / tot
    eps = 1e-10
    factor = 1.0 + 0.01 / (1.0 + 1 * 0)
    h = ((p0 + eps) * jnp.log(1.0 / (p0 + eps) + eps)
         + (p1 + eps) * jnp.log(1.0 / (p1 + eps) + eps))
    return h * factor


def _entropy_tc_body(xn_ref, xc_ref, hn_ref, hc_ref):
    hn_ref[...] = _entropy_cols(xn_ref[0, :], xn_ref[1, :])
    hc_ref[...] = _entropy_cols(xc_ref[0, :], xc_ref[1, :])


def _floor16(x):
    t = x.astype(jnp.int32).astype(jnp.float32)
    return t - jnp.where(x < t, 1.0, 0.0)


def _rmw_max(ref, idx, s):
    ks, vi = plsc.sort_key_val(s, idx)
    _, last = plsc.scan_count(vi)
    cur = plsc.load_gather(ref, [vi])
    plsc.store_scatter(ref, [vi], jnp.maximum(cur, ks), mask=last)


def _make_sc_kernel(n_nodes, n_edges):
    try:
        info = plsc.get_sparse_core_info()
        ns = info.num_subcores
    except ValueError:
        ns = 16
    nw = ns
    slc = ((n_nodes + ns * _L - 1) // (ns * _L)) * _L
    n_pad = ns * slc
    chunk = ((n_edges + nw * _L - 1) // (nw * _L)) * _L
    tail = n_edges - (nw - 1) * chunk
    assert tail > 0 and tail % _L == 0 and chunk % 32 == 0
    assert n_pad % 128 == 0 and slc % _L == 0
    mesh = plsc.VectorSubcoreMesh(core_axis_name="c", subcore_axis_name="s",
                                  num_cores=1)

    @functools.partial(
        pl.kernel,
        out_type=jax.ShapeDtypeStruct((n_edges,), jnp.float32),
        mesh=mesh,
        compiler_params=pltpu.CompilerParams(needs_layout_passes=False),
        scratch_types=[
            pltpu.VMEM((n_pad,), jnp.float32),
            pltpu.VMEM((n_pad,), jnp.float32),
            pltpu.VMEM((chunk,), jnp.int32),
            pltpu.VMEM((chunk,), jnp.int32),
            pltpu.VMEM((chunk,), jnp.float32),
            pltpu.VMEM((chunk,), jnp.float32),
            pltpu.VMEM_SHARED((ns * n_pad,), jnp.float32),
            pltpu.VMEM_SHARED((n_pad,), jnp.float32),
        ],
    )
    def sc_all(hn, hc, ei, out, h_v, nm_v, src_v, dst_v, hc_v, sc_v,
               partials, global_nm):
        sid = lax.axis_index("s")
        wid = sid
        base = wid * chunk

        pltpu.sync_copy(hn, h_v.at[pl.ds(0, n_nodes)])
        pltpu.sync_copy(ei.at[pl.ds(base, chunk)], src_v)

        @pl.when(wid < nw - 1)
        def _():
            pltpu.sync_copy(ei.at[pl.ds(n_edges + base, chunk)], dst_v)
            pltpu.sync_copy(hc.at[pl.ds(base, chunk)], hc_v)

        @pl.when(wid == nw - 1)
        def _():
            pltpu.sync_copy(ei.at[pl.ds(n_edges + base, tail)],
                            dst_v.at[pl.ds(0, tail)])
            pltpu.sync_copy(hc.at[pl.ds(base, tail)],
                            hc_v.at[pl.ds(0, tail)])
            izeros = jnp.zeros((_L,), jnp.int32)
            for u in range((chunk - tail) // _L):
                dst_v[pl.ds(tail + u * _L, _L)] = izeros

        zeros = jnp.zeros((_L,), jnp.float32)

        def zero_body(j, _):
            for u in range(8):
                nm_v[pl.ds(j * 8 * _L + u * _L, _L)] = zeros
            return 0

        lax.fori_loop(0, n_pad // (8 * _L), zero_body, 0)

        iota = lax.iota(jnp.int32, _L)

        def edge_body(j, _):
            for u in range(2):
                off = (j * 2 + u) * _L
                sl = pl.ds(off, _L)
                si = src_v[sl]
                di = dst_v[sl]
                hcv = hc_v[sl]
                hs = plsc.load_gather(h_v, [si])
                hd = plsc.load_gather(h_v, [di])
                a = hs - hcv
                b = hd - hcv
                fa = _floor16(a)
                fb = _floor16(b)
                s = (2.0 + a) * (2.0 + b) * ((1.0 + fa) * (1.0 + fb))
                lane = base + off + iota
                s = jnp.where(lane < n_edges, s, 0.0)
                sc_v[sl] = s
                _rmw_max(nm_v, si, s)
                _rmw_max(nm_v, di, s)
            return 0

        lax.fori_loop(0, chunk // (2 * _L), edge_body, 0)

        pltpu.sync_copy(nm_v, partials.at[pl.ds(sid * n_pad, n_pad)])
        plsc.subcore_barrier()
        for t in range(ns):
            pltpu.sync_copy(partials.at[pl.ds(t * n_pad + sid * slc, slc)],
                            h_v.at[pl.ds(t * slc, slc)])

        def red_body(j, _):
            off = j * _L
            acc = h_v[pl.ds(off, _L)]
            for t in range(1, ns):
                acc = jnp.maximum(acc, h_v[pl.ds(t * slc + off, _L)])
            nm_v[pl.ds(off, _L)] = acc
            return 0

        lax.fori_loop(0, slc // _L, red_body, 0)
        pltpu.sync_copy(nm_v.at[pl.ds(0, slc)],
                        global_nm.at[pl.ds(sid * slc, slc)])
        plsc.subcore_barrier()
        pltpu.sync_copy(global_nm, h_v.at[pl.ds(0, n_pad)])

        def sel_body(j, _):
            for u in range(2):
                sl = pl.ds((j * 2 + u) * _L, _L)
                s = sc_v[sl]
                ms = plsc.load_gather(h_v, [src_v[sl]])
                md = plsc.load_gather(h_v, [dst_v[sl]])
                keep = (s > 0.0) & (s >= ms) & (s >= md)
                sc_v[sl] = jnp.where(keep, s, 0.0)
            return 0

        lax.fori_loop(0, chunk // (2 * _L), sel_body, 0)

        @pl.when(wid < nw - 1)
        def _():
            pltpu.sync_copy(sc_v, out.at[pl.ds(base, chunk)])

        @pl.when(wid == nw - 1)
        def _():
            pltpu.sync_copy(sc_v.at[pl.ds(0, tail)], out.at[pl.ds(base, tail)])

    return sc_all


@jax.jit
def kernel(node_logits, comb_logits, edge_index):
    n_nodes = node_logits.shape[0]
    n_edges = comb_logits.shape[0]
    sc_all = _make_sc_kernel(n_nodes, n_edges)

    xn = node_logits.T
    xc = comb_logits.T
    hn, hc = pl.pallas_call(
        _entropy_tc_body,
        out_shape=(
            jax.ShapeDtypeStruct((n_nodes,), jnp.float32),
            jax.ShapeDtypeStruct((n_edges,), jnp.float32),
        ),
    )(xn, xc)
    ei = edge_index.reshape(2 * n_edges)

    return sc_all(hn, hc, ei)


# speedup vs baseline: 2.7121x; 1.0841x over previous
import functools

import jax
import jax.numpy as jnp
from jax import lax
from jax.experimental import pallas as pl
from jax.experimental.pallas import tpu as pltpu
from jax.experimental.pallas import tpu_sc as plsc

_L = 16


def _entropy_cols(l0, l1):
    m = jnp.maximum(l0, l1)
    e0 = jnp.exp(l0 - m)
    e1 = jnp.exp(l1 - m)
    tot = e0 + e1
    p0 = e0 / tot
    p1 = e1 / tot
    eps = 1e-10
    factor = 1.0 + 0.01 / (1.0 + 1 * 0)
    h = ((p0 + eps) * jnp.log(1.0 / (p0 + eps) + eps)
         + (p1 + eps) * jnp.log(1.0 / (p1 + eps) + eps))
    return h * factor


def _entropy_tc_body(xn_ref, xc_ref, hn_ref, hc_ref):
    hn_ref[...] = _entropy_cols(xn_ref[0, :], xn_ref[1, :])
    hc_ref[...] = _entropy_cols(xc_ref[0, :], xc_ref[1, :])


def _floor16(x):
    t = x.astype(jnp.int32).astype(jnp.float32)
    return t - jnp.where(x < t, 1.0, 0.0)


def _rmw_max(ref, idx, s):
    ks, vi = plsc.sort_key_val(s, idx)
    _, last = plsc.scan_count(vi)
    cur = plsc.load_gather(ref, [vi])
    plsc.store_scatter(ref, [vi], jnp.maximum(cur, ks), mask=last)


def _make_sc_kernel(n_nodes, n_edges):
    try:
        info = plsc.get_sparse_core_info()
        ns = info.num_subcores
    except ValueError:
        ns = 16
    nw = ns
    slc = ((n_nodes + ns * _L - 1) // (ns * _L)) * _L
    n_pad = ns * slc
    chunk = ((n_edges + nw * _L - 1) // (nw * _L)) * _L
    tail = n_edges - (nw - 1) * chunk
    assert tail > 0 and tail % _L == 0 and chunk % 32 == 0
    assert n_pad % 128 == 0 and slc % _L == 0
    mesh = plsc.VectorSubcoreMesh(core_axis_name="c", subcore_axis_name="s",
                                  num_cores=1)

    @functools.partial(
        pl.kernel,
        out_type=jax.ShapeDtypeStruct((n_edges,), jnp.float32),
        mesh=mesh,
        compiler_params=pltpu.CompilerParams(needs_layout_passes=False),
        scratch_types=[
            pltpu.VMEM((n_pad,), jnp.float32),
            pltpu.VMEM((n_pad,), jnp.float32),
            pltpu.VMEM((chunk,), jnp.int32),
            pltpu.VMEM((chunk,), jnp.int32),
            pltpu.VMEM((chunk,), jnp.float32),
            pltpu.VMEM((chunk,), jnp.float32),
            pltpu.VMEM_SHARED((ns * n_pad,), jnp.float32),
            pltpu.VMEM_SHARED((n_pad,), jnp.float32),
            pltpu.SemaphoreType.DMA,
        ],
    )
    def sc_all(hn, hc, ei, out, h_v, nm_v, src_v, dst_v, hc_v, sc_v,
               partials, global_nm, sem):
        sid = lax.axis_index("s")
        wid = sid
        base = wid * chunk

        h_copy = pltpu.async_copy(hn, h_v.at[pl.ds(0, n_nodes)], sem)

        zeros = jnp.zeros((_L,), jnp.float32)

        def zero_body(j, _):
            for u in range(8):
                nm_v[pl.ds(j * 8 * _L + u * _L, _L)] = zeros
            return 0

        lax.fori_loop(0, n_pad // (8 * _L), zero_body, 0)

        pltpu.sync_copy(ei.at[pl.ds(base, chunk)], src_v)

        @pl.when(wid < nw - 1)
        def _():
            pltpu.sync_copy(ei.at[pl.ds(n_edges + base, chunk)], dst_v)
            pltpu.sync_copy(hc.at[pl.ds(base, chunk)], hc_v)

        @pl.when(wid == nw - 1)
        def _():
            pltpu.sync_copy(ei.at[pl.ds(n_edges + base, tail)],
                            dst_v.at[pl.ds(0, tail)])
            pltpu.sync_copy(hc.at[pl.ds(base, tail)],
                            hc_v.at[pl.ds(0, tail)])
            izeros = jnp.zeros((_L,), jnp.int32)
            for u in range((chunk - tail) // _L):
                dst_v[pl.ds(tail + u * _L, _L)] = izeros
        h_copy.wait()

        iota = lax.iota(jnp.int32, _L)

        def edge_body(j, _):
            for u in range(2):
                off = (j * 2 + u) * _L
                sl = pl.ds(off, _L)
                si = src_v[sl]
                di = dst_v[sl]
                hcv = hc_v[sl]
                hs = plsc.load_gather(h_v, [si])
                hd = plsc.load_gather(h_v, [di])
                a = hs - hcv
                b = hd - hcv
                fa = _floor16(a)
                fb = _floor16(b)
                s = (2.0 + a) * (2.0 + b) * ((1.0 + fa) * (1.0 + fb))
                lane = base + off + iota
                s = jnp.where(lane < n_edges, s, 0.0)
                sc_v[sl] = s
                _rmw_max(nm_v, si, s)
                _rmw_max(nm_v, di, s)
            return 0

        lax.fori_loop(0, chunk // (2 * _L), edge_body, 0)

        pltpu.sync_copy(nm_v, partials.at[pl.ds(sid * n_pad, n_pad)])
        plsc.subcore_barrier()
        red_copy = None
        for t in range(ns):
            red_copy = pltpu.async_copy(
                partials.at[pl.ds(t * n_pad + sid * slc, slc)],
                h_v.at[pl.ds(t * slc, slc)], sem)
        for t in range(ns):
            red_copy.wait()

        def red_body(j, _):
            off = j * _L
            acc = h_v[pl.ds(off, _L)]
            for t in range(1, ns):
                acc = jnp.maximum(acc, h_v[pl.ds(t * slc + off, _L)])
            nm_v[pl.ds(off, _L)] = acc
            return 0

        lax.fori_loop(0, slc // _L, red_body, 0)
        pltpu.sync_copy(nm_v.at[pl.ds(0, slc)],
                        global_nm.at[pl.ds(sid * slc, slc)])
        plsc.subcore_barrier()
        pltpu.sync_copy(global_nm, h_v.at[pl.ds(0, n_pad)])

        def sel_body(j, _):
            for u in range(2):
                sl = pl.ds((j * 2 + u) * _L, _L)
                s = sc_v[sl]
                ms = plsc.load_gather(h_v, [src_v[sl]])
                md = plsc.load_gather(h_v, [dst_v[sl]])
                keep = (s > 0.0) & (s >= ms) & (s >= md)
                sc_v[sl] = jnp.where(keep, s, 0.0)
            return 0

        lax.fori_loop(0, chunk // (2 * _L), sel_body, 0)

        @pl.when(wid < nw - 1)
        def _():
            pltpu.sync_copy(sc_v, out.at[pl.ds(base, chunk)])

        @pl.when(wid == nw - 1)
        def _():
            pltpu.sync_copy(sc_v.at[pl.ds(0, tail)], out.at[pl.ds(base, tail)])

    return sc_all


@jax.jit
def kernel(node_logits, comb_logits, edge_index):
    n_nodes = node_logits.shape[0]
    n_edges = comb_logits.shape[0]
    sc_all = _make_sc_kernel(n_nodes, n_edges)

    xn = node_logits.T
    xc = comb_logits.T
    hn, hc = pl.pallas_call(
        _entropy_tc_body,
        out_shape=(
            jax.ShapeDtypeStruct((n_nodes,), jnp.float32),
            jax.ShapeDtypeStruct((n_edges,), jnp.float32),
        ),
    )(xn, xc)
    ei = edge_index.reshape(2 * n_edges)
    return sc_all(hn, hc, ei)
